# baseline (speedup 1.0000x reference)
import jax
import jax.numpy as jnp
from jax import lax
from jax.experimental import pallas as pl
from jax.experimental.pallas import tpu as pltpu
from jax.experimental.pallas import tpu_sc as plsc

_NQ = 1000
_NQP = 1008
_NT = 64
_NC = 91
_BIG = 2 ** 30


def _cost_kernel(labels_ref, logits_ref, pred_ref, tboxt_ref,
                 cost_ref, cv_ref, cr_ref):
    bq = logits_ref.shape[0]
    bs = bq // _NQ
    onehot = (lax.broadcasted_iota(jnp.int32, (_NC, _NT), 0)
              == labels_ref[...]).astype(jnp.float32)
    gathered = jax.lax.dot_general(
        logits_ref[...], onehot, (((1,), (0,)), ((), ())),
        precision=jax.lax.Precision.HIGHEST,
        preferred_element_type=jnp.float32)
    cost_class = -jax.nn.sigmoid(gathered)

    pb = pred_ref[...]
    q_cx, q_cy = pb[:, 0:1], pb[:, 1:2]
    q_w, q_h = pb[:, 2:3], pb[:, 3:4]
    tb = tboxt_ref[...]
    t_cx, t_cy = tb[0:1, :], tb[1:2, :]
    t_w, t_h = tb[2:3, :], tb[3:4, :]

    cost_bbox = ((jnp.abs(q_cx - t_cx) + jnp.abs(q_cy - t_cy))
                 + jnp.abs(q_w - t_w)) + jnp.abs(q_h - t_h)

    qx0, qy0 = q_cx - 0.5 * q_w, q_cy - 0.5 * q_h
    qx1, qy1 = q_cx + 0.5 * q_w, q_cy + 0.5 * q_h
    tx0, ty0 = t_cx - 0.5 * t_w, t_cy - 0.5 * t_h
    tx1, ty1 = t_cx + 0.5 * t_w, t_cy + 0.5 * t_h

    area_q = (qx1 - qx0) * (qy1 - qy0)
    area_t = (tx1 - tx0) * (ty1 - ty0)
    wx = jnp.maximum(jnp.minimum(qx1, tx1) - jnp.maximum(qx0, tx0), 0.0)
    wy = jnp.maximum(jnp.minimum(qy1, ty1) - jnp.maximum(qy0, ty0), 0.0)
    inter = wx * wy
    union = area_q + area_t - inter
    iou = inter / union
    ex = jnp.maximum(qx1, tx1) - jnp.minimum(qx0, tx0)
    ey = jnp.maximum(qy1, ty1) - jnp.minimum(qy0, ty0)
    area_e = jnp.maximum(ex, 0.0) * jnp.maximum(ey, 0.0)
    cost_giou = -(iou - (area_e - union) / area_e)

    cm = (cost_bbox + cost_class) + cost_giou
    cost_ref[...] = cm
    rloc = lax.broadcasted_iota(jnp.int32, (_NQ, _NT), 0)
    for b in range(bs):
        seg = cm[b * _NQ:(b + 1) * _NQ, :]
        m = jnp.min(seg, axis=0, keepdims=True)
        cv_ref[b:b + 1, :] = m
        cr_ref[b:b + 1, :] = jnp.min(jnp.where(seg == m, rloc, _BIG),
                                     axis=0, keepdims=True)


def _sc_greedy(cost_hbm, cv_hbm, cr_hbm, rows_hbm, cols_hbm,
               w_v, cv_v, cr_v, rows_v, cols_v):
    b = lax.axis_index("s") * 2 + lax.axis_index("c")
    nb = cv_hbm.shape[0]
    iota16 = lax.iota(jnp.int32, 16)
    lane0 = iota16 == 0
    inf = jnp.float32(jnp.inf)
    inf16 = jnp.full((16,), jnp.inf, jnp.float32)

    def spl_i(x):
        return jnp.full((16,), x, jnp.int32)

    def spl_f(x):
        return jnp.full((16,), x, jnp.float32)

    def col_min(col):
        def body(k, carry):
            lmin, lrow = carry
            rows = k * 16 + iota16
            val = plsc.load_gather(w_v, [rows, spl_i(col)])
            lt = val < lmin
            return jnp.where(lt, val, lmin), jnp.where(lt, rows, lrow)

        lmin, lrow = lax.fori_loop(0, _NQP // 16, body,
                                   (spl_f(inf), spl_i(_BIG)))
        m2 = jnp.min(lmin)
        r2 = jnp.min(jnp.where(lmin == m2, lrow, _BIG))
        return m2, r2

    @pl.when(b < nb)
    def _():
        for r in range(_NQ, _NQP):
            for c4 in range(4):
                w_v[r, pl.ds(c4 * 16, 16)] = inf16
        pltpu.sync_copy(cost_hbm.at[pl.ds(b * _NQ, _NQ)],
                        w_v.at[pl.ds(0, _NQ)])
        pltpu.sync_copy(cv_hbm.at[b], cv_v)
        pltpu.sync_copy(cr_hbm.at[b], cr_v)
        cv = [cv_v[pl.ds(j * 16, 16)] for j in range(4)]
        cr = [cr_v[pl.ds(j * 16, 16)] for j in range(4)]

        def step(i, carry):
            cv0, cv1, cv2, cv3, cr0, cr1, cr2, cr3 = carry
            cvj = [cv0, cv1, cv2, cv3]
            crj = [cr0, cr1, cr2, cr3]
            vm = jnp.minimum(jnp.minimum(cvj[0], cvj[1]),
                             jnp.minimum(cvj[2], cvj[3]))
            m = jnp.min(vm)
            fm = spl_i(_BIG)
            for j in range(4):
                ci = j * 16 + iota16
                fm = jnp.minimum(
                    fm, jnp.where(cvj[j] == m, crj[j] * _NT + ci, _BIG))
            f = jnp.min(fm)
            r = f // _NT
            c = f - r * _NT

            plsc.store_scatter(rows_v, [spl_i(i)], spl_i(r), mask=lane0)
            plsc.store_scatter(cols_v, [spl_i(i)], spl_i(c), mask=lane0)
            for j in range(4):
                ci = j * 16 + iota16
                cvj[j] = jnp.where(ci == c, inf, cvj[j])
                plsc.store_scatter(w_v, [spl_i(r), ci], spl_f(inf))
            stale = [(crj[j] == r) & (cvj[j] < inf) for j in range(4)]
            n_stale = jnp.max(jnp.maximum(
                jnp.maximum(stale[0].astype(jnp.int32),
                            stale[1].astype(jnp.int32)),
                jnp.maximum(stale[2].astype(jnp.int32),
                            stale[3].astype(jnp.int32))))

            def rescan(args):
                cvj, crj, stale = list(args[0]), list(args[1]), args[2]
                for j in range(4):
                    def scond(s):
                        return jnp.max(s[2].astype(jnp.int32)) > 0

                    def sbody(s):
                        cvx, crx, mask = s
                        jj = plsc.all_reduce_ffs(mask)
                        jj = jnp.min(jj) if getattr(jj, "ndim", 0) else jj
                        m2, r2 = col_min(j * 16 + jj)
                        cvx = jnp.where(iota16 == jj, m2, cvx)
                        crx = jnp.where(iota16 == jj, r2, crx)
                        return cvx, crx, mask & (iota16 != jj)

                    cvj[j], crj[j], _ = lax.while_loop(
                        scond, sbody, (cvj[j], crj[j], stale[j]))
                return tuple(cvj), tuple(crj)

            (cv0, cv1, cv2, cv3), (cr0, cr1, cr2, cr3) = lax.cond(
                n_stale > 0, rescan, lambda a: (a[0], a[1]),
                (tuple(cvj), tuple(crj), tuple(stale)))
            return cv0, cv1, cv2, cv3, cr0, cr1, cr2, cr3

        lax.fori_loop(0, _NT, step, tuple(cv) + tuple(cr))
        pltpu.sync_copy(rows_v, rows_hbm.at[b])
        pltpu.sync_copy(cols_v, cols_hbm.at[b])


def kernel(logits, pred_boxes, boxes, class_labels):
    bs = logits.shape[0]
    logits2d = logits.reshape(bs * _NQ, _NC)
    pred2d = pred_boxes.reshape(bs * _NQ, 4)
    tboxt = boxes[0].T
    labels = class_labels[0].reshape(1, _NT)

    cost, cv, cr = pl.pallas_call(
        _cost_kernel,
        out_shape=[
            jax.ShapeDtypeStruct((bs * _NQ, _NT), jnp.float32),
            jax.ShapeDtypeStruct((bs, _NT), jnp.float32),
            jax.ShapeDtypeStruct((bs, _NT), jnp.int32),
        ],
    )(labels, logits2d, pred2d, tboxt)

    mesh = plsc.VectorSubcoreMesh(core_axis_name="c", subcore_axis_name="s")
    rows, cols = pl.kernel(
        _sc_greedy,
        out_type=[jax.ShapeDtypeStruct((bs, _NT), jnp.int32)] * 2,
        mesh=mesh,
        scratch_types=[
            pltpu.VMEM((_NQP, _NT), jnp.float32),
            pltpu.VMEM((_NT,), jnp.float32),
            pltpu.VMEM((_NT,), jnp.int32),
            pltpu.VMEM((_NT,), jnp.int32),
            pltpu.VMEM((_NT,), jnp.int32),
        ],
        compiler_params=pltpu.CompilerParams(needs_layout_passes=False),
    )(cost, cv, cr)
    return rows, cols


# speedup vs baseline: 1.2941x; 1.2941x over previous
import jax
import jax.numpy as jnp
from jax import lax
from jax.experimental import pallas as pl
from jax.experimental.pallas import tpu as pltpu
from jax.experimental.pallas import tpu_sc as plsc

_NQ = 1000
_NQP = 1024
_NT = 64
_NC = 91
_BIG = 2 ** 30


def _cost_kernel(labels_ref, logits_ref, pred_ref, tbox_ref,
                 cost_ref, cv_ref, cr_ref):
    onehot = (lax.broadcasted_iota(jnp.int32, (_NT, _NC), 1)
              == labels_ref[...]).astype(jnp.float32)
    gathered = jax.lax.dot_general(
        onehot, logits_ref[0], (((1,), (0,)), ((), ())),
        precision=jax.lax.Precision.HIGHEST,
        preferred_element_type=jnp.float32)
    cost_class = -jax.nn.sigmoid(gathered)

    pb = pred_ref[0]
    q_cx, q_cy = pb[0:1, :], pb[1:2, :]
    q_w, q_h = pb[2:3, :], pb[3:4, :]
    tb = tbox_ref[...]
    t_cx, t_cy = tb[:, 0:1], tb[:, 1:2]
    t_w, t_h = tb[:, 2:3], tb[:, 3:4]

    cost_bbox = ((jnp.abs(q_cx - t_cx) + jnp.abs(q_cy - t_cy))
                 + jnp.abs(q_w - t_w)) + jnp.abs(q_h - t_h)

    qx0, qy0 = q_cx - 0.5 * q_w, q_cy - 0.5 * q_h
    qx1, qy1 = q_cx + 0.5 * q_w, q_cy + 0.5 * q_h
    tx0, ty0 = t_cx - 0.5 * t_w, t_cy - 0.5 * t_h
    tx1, ty1 = t_cx + 0.5 * t_w, t_cy + 0.5 * t_h

    area_q = (qx1 - qx0) * (qy1 - qy0)
    area_t = (tx1 - tx0) * (ty1 - ty0)
    wx = jnp.maximum(jnp.minimum(qx1, tx1) - jnp.maximum(qx0, tx0), 0.0)
    wy = jnp.maximum(jnp.minimum(qy1, ty1) - jnp.maximum(qy0, ty0), 0.0)
    inter = wx * wy
    union = area_q + area_t - inter
    iou = inter / union
    ex = jnp.maximum(qx1, tx1) - jnp.minimum(qx0, tx0)
    ey = jnp.maximum(qy1, ty1) - jnp.minimum(qy0, ty0)
    area_e = jnp.maximum(ex, 0.0) * jnp.maximum(ey, 0.0)
    cost_giou = -(iou - (area_e - union) / area_e)

    cm = (cost_bbox + cost_class) + cost_giou
    riota = lax.broadcasted_iota(jnp.int32, (_NT, _NQP), 1)
    cm = jnp.where(riota >= _NQ, jnp.float32(jnp.inf), cm)
    cost_ref[0] = cm
    m = jnp.min(cm, axis=1, keepdims=True)
    cv_ref[0] = m
    cr_ref[0] = jnp.min(jnp.where(cm == m, riota, _BIG),
                        axis=1, keepdims=True)


def _sc_greedy(cost_hbm, cv_hbm, cr_hbm, rows_hbm, cols_hbm,
               w_v, cv_v, cr_v, rows_v, cols_v):
    b = lax.axis_index("s") * 2 + lax.axis_index("c")
    nb = cv_hbm.shape[0]
    iota16 = lax.iota(jnp.int32, 16)
    lane0 = iota16 == 0
    inf = jnp.float32(jnp.inf)

    def spl_i(x):
        return jnp.full((16,), x, jnp.int32)

    def spl_f(x):
        return jnp.full((16,), x, jnp.float32)

    def col_min(col):
        def body(k, carry):
            lmin, lrow = carry
            rows = k * 16 + iota16
            val = plsc.load_gather(w_v, [spl_i(col), rows])
            lt = val < lmin
            return jnp.where(lt, val, lmin), jnp.where(lt, rows, lrow)

        lmin, lrow = lax.fori_loop(0, _NQP // 16, body,
                                   (spl_f(inf), spl_i(_BIG)))
        m2 = jnp.min(lmin)
        r2 = jnp.min(jnp.where(lmin == m2, lrow, _BIG))
        return m2, r2

    @pl.when(b < nb)
    def _():
        pltpu.sync_copy(cost_hbm.at[b], w_v)
        pltpu.sync_copy(cv_hbm.at[b], cv_v)
        pltpu.sync_copy(cr_hbm.at[b], cr_v)
        cv = [cv_v[pl.ds(j * 16, 16)] for j in range(4)]
        cr = [cr_v[pl.ds(j * 16, 16)] for j in range(4)]

        def step(i, carry):
            cv0, cv1, cv2, cv3, cr0, cr1, cr2, cr3 = carry
            cvj = [cv0, cv1, cv2, cv3]
            crj = [cr0, cr1, cr2, cr3]
            vm = jnp.minimum(jnp.minimum(cvj[0], cvj[1]),
                             jnp.minimum(cvj[2], cvj[3]))
            m = jnp.min(vm)
            fm = spl_i(_BIG)
            for j in range(4):
                ci = j * 16 + iota16
                fm = jnp.minimum(
                    fm, jnp.where(cvj[j] == m, crj[j] * _NT + ci, _BIG))
            f = jnp.min(fm)
            r = f // _NT
            c = f - r * _NT

            plsc.store_scatter(rows_v, [spl_i(i)], spl_i(r), mask=lane0)
            plsc.store_scatter(cols_v, [spl_i(i)], spl_i(c), mask=lane0)
            for j in range(4):
                ci = j * 16 + iota16
                cvj[j] = jnp.where(ci == c, inf, cvj[j])
                plsc.store_scatter(w_v, [ci, spl_i(r)], spl_f(inf))
            stale = [(crj[j] == r) & (cvj[j] < inf) for j in range(4)]
            n_stale = jnp.max(jnp.maximum(
                jnp.maximum(stale[0].astype(jnp.int32),
                            stale[1].astype(jnp.int32)),
                jnp.maximum(stale[2].astype(jnp.int32),
                            stale[3].astype(jnp.int32))))

            def rescan(args):
                cvj, crj, stale = list(args[0]), list(args[1]), args[2]
                for j in range(4):
                    def scond(s):
                        return jnp.max(s[2].astype(jnp.int32)) > 0

                    def sbody(s):
                        cvx, crx, mask = s
                        jj = plsc.all_reduce_ffs(mask)
                        jj = jnp.min(jj) if getattr(jj, "ndim", 0) else jj
                        m2, r2 = col_min(j * 16 + jj)
                        cvx = jnp.where(iota16 == jj, m2, cvx)
                        crx = jnp.where(iota16 == jj, r2, crx)
                        return cvx, crx, mask & (iota16 != jj)

                    cvj[j], crj[j], _ = lax.while_loop(
                        scond, sbody, (cvj[j], crj[j], stale[j]))
                return tuple(cvj), tuple(crj)

            (cv0, cv1, cv2, cv3), (cr0, cr1, cr2, cr3) = lax.cond(
                n_stale > 0, rescan, lambda a: (a[0], a[1]),
                (tuple(cvj), tuple(crj), tuple(stale)))
            return cv0, cv1, cv2, cv3, cr0, cr1, cr2, cr3

        lax.fori_loop(0, _NT, step, tuple(cv) + tuple(cr))
        pltpu.sync_copy(rows_v, rows_hbm.at[b])
        pltpu.sync_copy(cols_v, cols_hbm.at[b])


def kernel(logits, pred_boxes, boxes, class_labels):
    bs = logits.shape[0]
    logits_t = jnp.pad(jnp.swapaxes(logits, 1, 2),
                       ((0, 0), (0, 0), (0, _NQP - _NQ)))
    pred_t = jnp.pad(jnp.swapaxes(pred_boxes, 1, 2),
                     ((0, 0), (0, 0), (0, _NQP - _NQ)))
    tbox = boxes[0]
    labels = class_labels[0].reshape(_NT, 1)

    cost, cv, cr = pl.pallas_call(
        _cost_kernel,
        grid=(bs,),
        in_specs=[
            pl.BlockSpec((_NT, 1), lambda b: (0, 0)),
            pl.BlockSpec((1, _NC, _NQP), lambda b: (b, 0, 0)),
            pl.BlockSpec((1, 4, _NQP), lambda b: (b, 0, 0)),
            pl.BlockSpec((_NT, 4), lambda b: (0, 0)),
        ],
        out_specs=[
            pl.BlockSpec((1, _NT, _NQP), lambda b: (b, 0, 0)),
            pl.BlockSpec((1, _NT, 1), lambda b: (b, 0, 0)),
            pl.BlockSpec((1, _NT, 1), lambda b: (b, 0, 0)),
        ],
        out_shape=[
            jax.ShapeDtypeStruct((bs, _NT, _NQP), jnp.float32),
            jax.ShapeDtypeStruct((bs, _NT, 1), jnp.float32),
            jax.ShapeDtypeStruct((bs, _NT, 1), jnp.int32),
        ],
    )(labels, logits_t, pred_t, tbox)

    mesh = plsc.VectorSubcoreMesh(core_axis_name="c", subcore_axis_name="s")
    rows, cols = pl.kernel(
        _sc_greedy,
        out_type=[jax.ShapeDtypeStruct((bs, _NT), jnp.int32)] * 2,
        mesh=mesh,
        scratch_types=[
            pltpu.VMEM((_NT, _NQP), jnp.float32),
            pltpu.VMEM((_NT,), jnp.float32),
            pltpu.VMEM((_NT,), jnp.int32),
            pltpu.VMEM((_NT,), jnp.int32),
            pltpu.VMEM((_NT,), jnp.int32),
        ],
        compiler_params=pltpu.CompilerParams(needs_layout_passes=False),
    )(cost, cv.reshape(bs, _NT), cr.reshape(bs, _NT))
    return rows, cols
